# Initial kernel scaffold; baseline (speedup 1.0000x reference)
import jax
import jax.numpy as jnp
from jax.experimental import pallas as pl


def kernel(params):
    raise NotImplementedError("write your pallas kernel here")


# speedup vs baseline: 2.9116x; 2.9116x over previous
import functools

import jax
import jax.numpy as jnp
import numpy as np
from jax import lax
from jax.experimental import pallas as pl
from jax.experimental.pallas import tpu as pltpu
from jax.experimental.pallas import tpu_sc as plsc

HH, WW, K = 128, 352, 25
NC, NS, L = 2, 16, 16
NW = NC * NS
SX = 2.0 / (WW - 1)
SY = 2.0 / (HH - 1)
MAGIC = np.float32(12582912.0)
NEG = np.float32(-1e30)
RSQRT_SEED = np.int32(0x5F3759DF)


L2E = np.float32(1.4426950408889634)
EC5 = np.float32(0.0013260914711281657)
EC4 = np.float32(0.009670180268585682)
EC3 = np.float32(0.055507123470306396)
EC2 = np.float32(0.2402222454547882)
EC1 = np.float32(0.6931470036506653)
EC0 = np.float32(1.0)


def _exp(t):
  z = jnp.maximum(t * L2E, -126.0)
  k = (z + MAGIC) - MAGIC
  f = z - k
  p = EC5
  for c in (EC4, EC3, EC2, EC1, EC0):
    p = p * f + c
  scale = plsc.bitcast(
      lax.shift_left(k.astype(jnp.int32) + 127, 23), jnp.float32)
  return p * scale


def _rcp(x):
  r = 1.0 / x
  r = r * (2.0 - x * r)
  r = r * (2.0 - x * r)
  return r


def _merge32(T0, T1, I0, I1, s, si):
  cs, ci = plsc.sort_key_val(s, si, descending=True)
  rc = lax.rev(cs, (0,))
  rci = lax.rev(ci, (0,))
  gt = T1 >= rc
  u = jnp.where(gt, T1, rc)
  ui = jnp.where(gt, I1, rci)
  us, uis = plsc.sort_key_val(u, ui, descending=True)
  ru = lax.rev(us, (0,))
  rui = lax.rev(uis, (0,))
  g0 = T0 >= ru
  v = jnp.where(g0, T0, ru)
  vi = jnp.where(g0, I0, rui)
  w = jnp.where(g0, ru, T0)
  wi = jnp.where(g0, rui, I0)
  nT0, nI0 = plsc.sort_key_val(v, vi, descending=True)
  nT1, nI1 = plsc.sort_key_val(w, wi, descending=True)
  return nT0, nT1, nI0, nI1


@functools.lru_cache(maxsize=None)
def _make_call(n_pad, rpw, ch):
  nch = rpw // ch
  ngrp = ch // L
  mesh = plsc.VectorSubcoreMesh(core_axis_name="c", subcore_axis_name="s")
  out_type = (
      jax.ShapeDtypeStruct((n_pad * 25,), jnp.float32),
      jax.ShapeDtypeStruct((n_pad * 25,), jnp.int32),
      jax.ShapeDtypeStruct((n_pad * 25,), jnp.int32),
  )
  scratch = [
      pltpu.VMEM((ch * 8,), jnp.float32),
      pltpu.VMEM((ch * 16,), jnp.float32),
      pltpu.VMEM((ch * 25,), jnp.float32),
      pltpu.VMEM((ch * 25,), jnp.int32),
      pltpu.VMEM((ch * 25,), jnp.int32),
  ]

  @functools.partial(pl.kernel, out_type=out_type, mesh=mesh,
                     scratch_types=scratch,
                     compiler_params=pltpu.CompilerParams(
                         needs_layout_passes=False))
  def body(params_hbm, w_hbm, x_hbm, y_hbm, pv, dv, wb, xb, yb):
    wid = lax.axis_index("s") * NC + lax.axis_index("c")
    lanei = lax.iota(jnp.int32, L)
    lanef = lanei.astype(jnp.float32)
    msk9 = lanei < 9

    def chunk_body(c, carry):
      base = wid * rpw + c * ch
      pltpu.sync_copy(params_hbm.at[pl.ds(base * 8, ch * 8)], pv)

      def prep(gi, carry2):
        rows = gi * L + lanei
        rows8 = rows * 8
        rows16 = rows * 16

        def col(j):
          return plsc.load_gather(pv, [rows8 + j])

        mx, my, vx, vy, cr = col(0), col(1), col(2), col(3), col(4)
        prod = vx * vy
        h = plsc.bitcast(
            RSQRT_SEED - lax.shift_right_arithmetic(
                plsc.bitcast(prod, jnp.int32), 1), jnp.float32)
        for _ in range(3):
          h = h * (1.5 - 0.5 * prod * h * h)
        cov = cr * (prod * h)
        det = prod - cov * cov
        rdet = _rcp(det)
        ca = -0.5 * (vy * rdet)
        cb = cov * rdet
        cc = -0.5 * (vx * rdet)
        xi = ((mx + 1.0) / 2.0 * (WW - 1) + MAGIC) - MAGIC
        yi = ((my + 1.0) / 2.0 * (HH - 1) + MAGIC) - MAGIC
        xi = jnp.clip(xi, 12.0, float(WW - 13))
        yi = jnp.clip(yi, 12.0, float(HH - 13))

        def putcol(j, val):
          plsc.store_scatter(dv, [rows16 + j], val)

        putcol(0, xi)
        putcol(1, yi)
        putcol(2, mx)
        putcol(3, my)
        putcol(4, ca)
        putcol(5, cb)
        putcol(6, cc)
        return carry2

      lax.fori_loop(0, ngrp, prep, 0)

      def row_body(r, carry3):
        drow = dv[pl.ds(r * 16, 16)]
        xi = drow[0]
        yi = drow[1]
        mx = drow[2]
        my = drow[3]
        ca = drow[4]
        cb = drow[5]
        cc = drow[6]
        dy0 = (yi + (lanef - 12.0)) * SY - 1.0 - my
        dy1 = (yi + (lanef + 4.0)) * SY - 1.0 - my
        ry0 = cc * dy0 * dy0
        ry1 = cc * dy1 * dy1

        def col_body(i, tk):
          T0, T1, I0, I1, thr = tk
          fi = i.astype(jnp.float32)
          dxi = (xi + (fi - 12.0)) * SX - 1.0 - mx
          p = ca * dxi * dxi
          q = cb * dxi
          s0 = q * dy0 + ry0 + p
          s1 = jnp.where(msk9, q * dy1 + ry1 + p, NEG)
          i0 = i * 25 + lanei
          i1 = i0 + 16

          def do0(a, b, c2, d):
            o0, o1, o2, o3 = _merge32(a, b, c2, d, s0, i0)
            return o0, o1, o2, o3, jnp.min(o1)

          def skip(a, b, c2, d):
            return a, b, c2, d, thr

          T0, T1, I0, I1, thr = lax.cond(
              jnp.any(s0 > thr), do0, skip, T0, T1, I0, I1)

          def do1(a, b, c2, d):
            o0, o1, o2, o3 = _merge32(a, b, c2, d, s1, i1)
            return o0, o1, o2, o3, jnp.min(o1)

          def skip1(a, b, c2, d):
            return a, b, c2, d, thr

          T0, T1, I0, I1, thr = lax.cond(
              jnp.any(s1 > thr), do1, skip1, T0, T1, I0, I1)
          return T0, T1, I0, I1, thr

        init = (jnp.full((L,), NEG), jnp.full((L,), NEG),
                jnp.zeros((L,), jnp.int32), jnp.zeros((L,), jnp.int32),
                NEG)
        T0, T1, I0, I1, _ = lax.fori_loop(0, 25, col_body, init)

        maxv = jnp.max(T0)
        e0 = _exp(T0 - maxv)
        e1 = jnp.where(msk9, _exp(T1 - maxv), 0.0)
        ssum = jnp.sum(e0) + jnp.sum(e1)
        rs = _rcp(jnp.broadcast_to(ssum, (L,)))
        w0 = e0 * rs
        w1 = e1 * rs
        ii0 = I0 // 25
        jj0 = I0 - ii0 * 25
        ii1 = I1 // 25
        jj1 = I1 - ii1 * 25
        xib = xi.astype(jnp.int32) - 12
        yib = yi.astype(jnp.int32) - 12
        off = r * 25 + lanei
        off1 = off + 16
        plsc.store_scatter(wb, [off], w0)
        plsc.store_scatter(wb, [off1], w1, mask=msk9)
        plsc.store_scatter(xb, [off], xib + ii0)
        plsc.store_scatter(xb, [off1], xib + ii1, mask=msk9)
        plsc.store_scatter(yb, [off], yib + jj0)
        plsc.store_scatter(yb, [off1], yib + jj1, mask=msk9)
        return carry3

      lax.fori_loop(0, ch, row_body, 0)
      pltpu.sync_copy(wb, w_hbm.at[pl.ds(base * 25, ch * 25)])
      pltpu.sync_copy(xb, x_hbm.at[pl.ds(base * 25, ch * 25)])
      pltpu.sync_copy(yb, y_hbm.at[pl.ds(base * 25, ch * 25)])
      return carry

    lax.fori_loop(0, nch, chunk_body, 0)

  return body


def kernel(params):
  n = params.shape[0]
  rpw = -(-n // NW)
  rpw = -(-rpw // 32) * 32
  ch = rpw
  while ch * 25 * 4 * 3 + ch * 64 > 400 * 1024 and ch % 32 == 0:
    ch //= 2
  assert ch % L == 0 and rpw % ch == 0
  n_pad = NW * rpw
  pp = jnp.pad(params.astype(jnp.float32), ((0, n_pad - n), (0, 3)))
  if n_pad > n:
    pp = pp.at[n:, 2:4].set(0.01)
  wf, xf, yf = _make_call(n_pad, rpw, ch)(pp.reshape(-1))
  return (wf.reshape(n_pad, 25)[:n],
          xf.reshape(n_pad, 25)[:n],
          yf.reshape(n_pad, 25)[:n])
